# baseline (speedup 1.0000x reference)
import jax
import jax.numpy as jnp
from jax.experimental import pallas as pl
from jax.experimental.pallas import tpu as pltpu

N_NODES = 10000
F_IN = 256
F_OUT = 256

BM = 200


def _gcn_kernel(adj_ref, x_ref, w_ref, b_ref, out_ref, s_ref):
    @pl.when(pl.program_id(0) == 0)
    def _make_support():
        s_ref[...] = jnp.dot(
            x_ref[...].astype(jnp.bfloat16),
            w_ref[...].astype(jnp.bfloat16),
            preferred_element_type=jnp.float32,
        ).astype(jnp.bfloat16)

    out_ref[...] = jnp.dot(
        adj_ref[...].astype(jnp.bfloat16),
        s_ref[...],
        preferred_element_type=jnp.float32,
    ) + b_ref[...]


def kernel(input, adj, W, b):
    b2d = b.reshape(1, F_OUT)
    return pl.pallas_call(
        _gcn_kernel,
        grid=(N_NODES // BM,),
        in_specs=[
            pl.BlockSpec((BM, N_NODES), lambda i: (i, 0)),
            pl.BlockSpec((N_NODES, F_IN), lambda i: (0, 0)),
            pl.BlockSpec((F_IN, F_OUT), lambda i: (0, 0)),
            pl.BlockSpec((1, F_OUT), lambda i: (0, 0)),
        ],
        out_specs=pl.BlockSpec((BM, F_OUT), lambda i: (i, 0)),
        out_shape=jax.ShapeDtypeStruct((N_NODES, F_OUT), jnp.float32),
        scratch_shapes=[pltpu.VMEM((N_NODES, F_OUT), jnp.bfloat16)],
        compiler_params=pltpu.CompilerParams(
            dimension_semantics=("arbitrary",),
        ),
    )(adj, input, W, b2d)


# speedup vs baseline: 1.0146x; 1.0146x over previous
import jax
import jax.numpy as jnp
from jax.experimental import pallas as pl
from jax.experimental.pallas import tpu as pltpu

N_NODES = 10000
F_IN = 256
F_OUT = 256

BM = 400


def _gcn_kernel(adj_ref, x_ref, w_ref, b_ref, out_ref, s_ref):
    @pl.when(pl.program_id(0) == 0)
    def _make_support():
        s_ref[...] = jnp.dot(
            x_ref[...], w_ref[...], preferred_element_type=jnp.float32,
        )

    out_ref[...] = jnp.dot(
        adj_ref[...], s_ref[...], preferred_element_type=jnp.float32,
    ) + b_ref[...]


def kernel(input, adj, W, b):
    b2d = b.reshape(1, F_OUT)
    return pl.pallas_call(
        _gcn_kernel,
        grid=(N_NODES // BM,),
        in_specs=[
            pl.BlockSpec((BM, N_NODES), lambda i: (i, 0)),
            pl.BlockSpec((N_NODES, F_IN), lambda i: (0, 0)),
            pl.BlockSpec((F_IN, F_OUT), lambda i: (0, 0)),
            pl.BlockSpec((1, F_OUT), lambda i: (0, 0)),
        ],
        out_specs=pl.BlockSpec((BM, F_OUT), lambda i: (i, 0)),
        out_shape=jax.ShapeDtypeStruct((N_NODES, F_OUT), jnp.float32),
        scratch_shapes=[pltpu.VMEM((N_NODES, F_OUT), jnp.float32)],
        compiler_params=pltpu.CompilerParams(
            dimension_semantics=("arbitrary",),
        ),
    )(adj, input, W, b2d)
